# Initial kernel scaffold; baseline (speedup 1.0000x reference)
import jax
import jax.numpy as jnp
from jax.experimental import pallas as pl


def kernel(x, edge_index, time_index, c1_W, c1_b, c2_W, c2_b, a_W1, a_b1, a_W2, a_b2):
    raise NotImplementedError("write your pallas kernel here")


# speedup vs baseline: 37.3331x; 37.3331x over previous
import functools

import jax
import jax.numpy as jnp
from jax import lax
from jax.experimental import pallas as pl
from jax.experimental.pallas import tpu as pltpu
from jax.experimental.pallas import tpu_sc as plsc

N = 50000
T = 4
D = 64
H = 32

NUM_TILES = 16
IDX_W = 128
ROWS_PER_CHUNK = 4
CHUNK = IDX_W * ROWS_PER_CHUNK
ACC_PER_TILE = 3128
ACC_ROWS = NUM_TILES * ACC_PER_TILE
OUT_PER_TILE = 3128
OUT_LAST = N - 15 * OUT_PER_TILE

BN = 2000
GRID = N // BN


def _mm_first_body(x_ref, wlo_ref, whi_ref, olo_ref, ohi_ref):
    h = x_ref[...]
    olo_ref[...] = jnp.dot(h, wlo_ref[...], preferred_element_type=jnp.float32)
    ohi_ref[...] = jnp.dot(h, whi_ref[...], preferred_element_type=jnp.float32)


def _mm_mid_body(lo_ref, hi_ref, b_ref, wlo_ref, whi_ref, olo_ref, ohi_ref):
    acc = jnp.concatenate([lo_ref[...], hi_ref[...]], axis=1)
    h = jax.nn.relu(acc + b_ref[...])
    olo_ref[...] = jnp.dot(h, wlo_ref[...], preferred_element_type=jnp.float32)
    ohi_ref[...] = jnp.dot(h, whi_ref[...], preferred_element_type=jnp.float32)


def _mm_adv_body(lo_ref, hi_ref, b_ref, wlo_ref, whi_ref,
                 aw1_ref, ab1_ref, aw2_ref, ab2_ref,
                 olo_ref, ohi_ref, adv_ref):
    acc = jnp.concatenate([lo_ref[...], hi_ref[...]], axis=1)
    h = jax.nn.relu(acc + b_ref[...])
    olo_ref[...] = jnp.dot(h, wlo_ref[...], preferred_element_type=jnp.float32)
    ohi_ref[...] = jnp.dot(h, whi_ref[...], preferred_element_type=jnp.float32)
    t1 = jax.nn.relu(jnp.dot(h, aw1_ref[...], preferred_element_type=jnp.float32)
                     + ab1_ref[...])
    adv_ref[...] = jnp.dot(t1, aw2_ref[...], preferred_element_type=jnp.float32) \
        + ab2_ref[...]


def _final_body(lo_ref, hi_ref, b_ref, adv_ref, out_ref):
    acc = jnp.concatenate([lo_ref[...], hi_ref[...]], axis=1)
    out_ref[...] = jax.nn.relu(acc + b_ref[...]) + adv_ref[...]


def _row_spec(w):
    return pl.BlockSpec((BN, w), lambda i: (i, 0))


def _full_spec(r, c):
    return pl.BlockSpec((r, c), lambda i: (0, 0))


def _mm_first(x, wlo, whi):
    return pl.pallas_call(
        _mm_first_body,
        grid=(GRID,),
        in_specs=[_row_spec(D), _full_spec(D, T * H), _full_spec(D, T * H)],
        out_specs=[_row_spec(T * H), _row_spec(T * H)],
        out_shape=[jax.ShapeDtypeStruct((N, T * H), jnp.float32)] * 2,
    )(x, wlo, whi)


def _mm_mid(lo, hi, b, wlo, whi):
    return pl.pallas_call(
        _mm_mid_body,
        grid=(GRID,),
        in_specs=[_row_spec(H), _row_spec(H), _full_spec(1, D),
                  _full_spec(D, T * H), _full_spec(D, T * H)],
        out_specs=[_row_spec(T * H), _row_spec(T * H)],
        out_shape=[jax.ShapeDtypeStruct((N, T * H), jnp.float32)] * 2,
    )(lo, hi, b, wlo, whi)


def _mm_adv(lo, hi, b, wlo, whi, aw1, ab1, aw2, ab2):
    return pl.pallas_call(
        _mm_adv_body,
        grid=(GRID,),
        in_specs=[_row_spec(H), _row_spec(H), _full_spec(1, D),
                  _full_spec(D, T * H), _full_spec(D, T * H),
                  _full_spec(D, 128), _full_spec(1, 128),
                  _full_spec(128, D), _full_spec(1, D)],
        out_specs=[_row_spec(T * H), _row_spec(T * H), _row_spec(D)],
        out_shape=[jax.ShapeDtypeStruct((N, T * H), jnp.float32),
                   jax.ShapeDtypeStruct((N, T * H), jnp.float32),
                   jax.ShapeDtypeStruct((N, D), jnp.float32)],
    )(lo, hi, b, wlo, whi, aw1, ab1, aw2, ab2)


def _final(lo, hi, b, adv):
    return pl.pallas_call(
        _final_body,
        grid=(GRID,),
        in_specs=[_row_spec(H), _row_spec(H), _full_spec(1, D), _row_spec(D)],
        out_specs=_row_spec(D),
        out_shape=jax.ShapeDtypeStruct((N, D), jnp.float32),
    )(lo, hi, b, adv)


def _make_sc_pass(chunks_per_tile, idx_rows):
    rows_per_tile = idx_rows // NUM_TILES
    mesh = plsc.VectorSubcoreMesh(core_axis_name="c", subcore_axis_name="s")

    @functools.partial(
        pl.kernel,
        mesh=mesh,
        compiler_params=pltpu.CompilerParams(use_tc_tiling_on_sc=False),
        out_type=(jax.ShapeDtypeStruct((N, H), jnp.float32),
                  jax.ShapeDtypeStruct((N, H), jnp.float32)),
        scratch_types=[
            pltpu.VMEM((ROWS_PER_CHUNK, IDX_W), jnp.int32),
            pltpu.VMEM((ROWS_PER_CHUNK, IDX_W), jnp.int32),
            pltpu.VMEM((CHUNK, H), jnp.float32),
            pltpu.VMEM_SHARED((ACC_ROWS, H), jnp.float32),
            pltpu.SemaphoreType.DMA,
            pltpu.SemaphoreType.DMA,
        ],
    )
    def sc_pass(gidx_hbm, didx_hbm, glo_hbm, ghi_hbm, out_lo, out_hi,
                gidx_v, didx_v, rows_v, acc, gsem, ssem):
        c = lax.axis_index("c")
        s = lax.axis_index("s")

        zv = jnp.zeros((16,), jnp.float32)

        def zrow(i, carry):
            rows_v[i, pl.ds(0, 16)] = zv
            rows_v[i, pl.ds(16, 16)] = zv
            return carry

        lax.fori_loop(0, CHUNK, zrow, 0)
        base = s * ACC_PER_TILE
        nfull = ACC_PER_TILE // CHUNK
        rem = ACC_PER_TILE - nfull * CHUNK

        def zcopy(k, carry):
            pltpu.sync_copy(rows_v, acc.at[pl.ds(base + k * CHUNK, CHUNK)])
            return carry

        lax.fori_loop(0, nfull, zcopy, 0)
        if rem:
            pltpu.sync_copy(rows_v.at[pl.ds(0, rem)],
                            acc.at[pl.ds(base + nfull * CHUNK, rem)])
        plsc.subcore_barrier()

        def run(g_hbm, out_hbm):
            def chunk_body(ci, carry):
                row0 = s * rows_per_tile + ci * ROWS_PER_CHUNK
                pltpu.sync_copy(gidx_hbm.at[pl.ds(row0, ROWS_PER_CHUNK)], gidx_v)
                pltpu.sync_copy(didx_hbm.at[pl.ds(row0, ROWS_PER_CHUNK)], didx_v)
                gd = [pltpu.async_copy(g_hbm.at[gidx_v.at[j]],
                                       rows_v.at[pl.ds(j * IDX_W, IDX_W)], gsem)
                      for j in range(ROWS_PER_CHUNK)]
                for dcp in gd:
                    dcp.wait()
                sd = [pltpu.async_copy(rows_v.at[pl.ds(j * IDX_W, IDX_W)],
                                       acc.at[didx_v.at[j]], ssem, add=True)
                      for j in range(ROWS_PER_CHUNK)]
                for dcp in sd:
                    dcp.wait()
                return carry

            lax.fori_loop(0, chunks_per_tile, chunk_body, 0)
            plsc.subcore_barrier()
            ob = s * OUT_PER_TILE

            @pl.when(s < 15)
            def _():
                pltpu.sync_copy(acc.at[pl.ds(ob, OUT_PER_TILE)],
                                out_hbm.at[pl.ds(ob, OUT_PER_TILE)])

            @pl.when(s == 15)
            def _():
                pltpu.sync_copy(acc.at[pl.ds(15 * OUT_PER_TILE, OUT_LAST)],
                                out_hbm.at[pl.ds(15 * OUT_PER_TILE, OUT_LAST)])

        @pl.when(c == 0)
        def _():
            run(glo_hbm, out_lo)

        @pl.when(c == 1)
        def _():
            run(ghi_hbm, out_hi)

    return sc_pass


def _split_w(W):
    r = W.reshape(D, T, D).transpose(2, 1, 0)
    wlo = r[:, :, :H].reshape(D, T * H)
    whi = r[:, :, H:].reshape(D, T * H)
    return wlo, whi


def kernel(x, edge_index, time_index, c1_W, c1_b, c2_W, c2_b,
           a_W1, a_b1, a_W2, a_b2):
    E = edge_index.shape[1]
    etot = E + N
    loops = jnp.arange(N, dtype=jnp.int32)
    srcf = jnp.concatenate([edge_index[0].astype(jnp.int32), loops])
    dstf = jnp.concatenate([edge_index[1].astype(jnp.int32), loops])
    gidx = srcf * T + time_index.astype(jnp.int32)

    chunks_per_tile = -(-etot // (NUM_TILES * CHUNK))
    epad = chunks_per_tile * NUM_TILES * CHUNK
    pad = epad - etot
    gidx_p = jnp.concatenate([gidx, jnp.zeros((pad,), jnp.int32)])
    didx_p = jnp.concatenate([dstf, jnp.full((pad,), N, jnp.int32)])
    idx_rows = epad // IDX_W
    gidx_p = gidx_p.reshape(idx_rows, IDX_W)
    didx_p = didx_p.reshape(idx_rows, IDX_W)

    w1lo, w1hi = _split_w(c1_W)
    w2lo, w2hi = _split_w(c2_W)
    b1 = c1_b.reshape(1, D)
    b2 = c2_b.reshape(1, D)
    aw1 = a_W1.T
    ab1 = a_b1.reshape(1, 128)
    aw2 = a_W2.T
    ab2 = a_b2.reshape(1, D)

    sc_pass = _make_sc_pass(chunks_per_tile, idx_rows)

    g1lo, g1hi = _mm_first(x, w1lo, w1hi)
    acc1lo, acc1hi = sc_pass(gidx_p, didx_p,
                             g1lo.reshape(N * T, H), g1hi.reshape(N * T, H))

    g2lo, g2hi = _mm_mid(acc1lo, acc1hi, b1, w2lo, w2hi)
    acc2lo, acc2hi = sc_pass(gidx_p, didx_p,
                             g2lo.reshape(N * T, H), g2hi.reshape(N * T, H))

    g3lo, g3hi, adv = _mm_adv(acc2lo, acc2hi, b2, w1lo, w1hi,
                              aw1, ab1, aw2, ab2)
    acc3lo, acc3hi = sc_pass(gidx_p, didx_p,
                             g3lo.reshape(N * T, H), g3hi.reshape(N * T, H))

    return _final(acc3lo, acc3hi, b1, adv)
